# Initial kernel scaffold; baseline (speedup 1.0000x reference)
import jax
import jax.numpy as jnp
from jax.experimental import pallas as pl


def kernel(wav, offsets):
    raise NotImplementedError("write your pallas kernel here")


# speedup vs baseline: 1.7761x; 1.7761x over previous
import functools

import jax
import jax.numpy as jnp
from jax import lax
from jax.experimental import pallas as pl
from jax.experimental.pallas import tpu as pltpu
from jax.experimental.pallas import tpu_sc as plsc

_SHIFT = 44100
_B, _S, _C, _T = 8, 4, 2, 441000
_L = _T - _SHIFT
_NW = 32
_IN_ROW = _C * _T
_OUT_ROW = _C * _L
_CH = 32704
_NFULL = _OUT_ROW // _CH
_TAIL = _OUT_ROW - _NFULL * _CH
_JSTR = _L // _CH
_PA = _L - _JSTR * _CH
_PB = _CH - _PA
_BUF = _CH + 48


def _round16(x):
    return (x + 15) // 16 * 16


def _shift_loop(dst, src, rem, length, dst_base):
    nit = (length + 63) // 64

    def body(i, _):
        base = i * 64
        for k in range(4):
            dst[pl.ds(dst_base + base + k * 16, 16)] = (
                src[pl.ds(rem + base + k * 16, 16)])
        return 0

    lax.fori_loop(0, nit, body, 0)


def _impl(wav_flat, offs):
    mesh = plsc.VectorSubcoreMesh(core_axis_name="c", subcore_axis_name="s")

    @functools.partial(
        pl.kernel,
        mesh=mesh,
        out_type=jax.ShapeDtypeStruct((_NW * _OUT_ROW,), jnp.float32),
        scratch_types=[
            pltpu.VMEM((48,), jnp.int32),
            pltpu.VMEM((_BUF,), jnp.float32),
            pltpu.VMEM((_BUF,), jnp.float32),
        ],
    )
    def k(wav_hbm, off_hbm, out_hbm, off_v, slot, strb):
        cid = lax.axis_index("c")
        sid = lax.axis_index("s")
        w = cid * 16 + sid

        pltpu.sync_copy(off_hbm.at[pl.ds(0, 32)], off_v.at[pl.ds(0, 32)])
        off = off_v[pl.ds(w, 16)][0]

        r0 = off % 8
        a0 = off - r0
        r1 = (off + _SHIFT) % 8
        a1 = off + _SHIFT - r1

        wbase_in = w * _IN_ROW
        wbase_out = w * _OUT_ROW

        for j in range(_NFULL + 1):
            u0 = j * _CH
            length = _CH if j < _NFULL else _TAIL
            glen = _round16(length + 8)
            if j != _JSTR:
                if j < _JSTR:
                    start, rem = wbase_in + a0 + u0, r0
                else:
                    start, rem = wbase_in + a1 + u0, r1
                pltpu.sync_copy(
                    wav_hbm.at[pl.ds(pl.multiple_of(start, 8), glen)],
                    slot.at[pl.ds(0, glen)])
                _shift_loop(slot, slot, rem, length, 0)
            else:
                ga = _round16(_PA + 8)
                gb = _round16(_PB + 8)
                pltpu.sync_copy(
                    wav_hbm.at[pl.ds(pl.multiple_of(wbase_in + a0 + u0, 8),
                                     ga)],
                    slot.at[pl.ds(0, ga)])
                pltpu.sync_copy(
                    wav_hbm.at[pl.ds(pl.multiple_of(wbase_in + a0 + _T, 8),
                                     gb)],
                    strb.at[pl.ds(0, gb)])
                _shift_loop(slot, slot, r0, _PA, 0)
                _shift_loop(slot, strb, r0, _PB, _PA)
            pltpu.sync_copy(
                slot.at[pl.ds(0, length)],
                out_hbm.at[pl.ds(wbase_out + u0, length)])

    return k(wav_flat, offs)


def kernel(wav, offsets):
    wav_flat = wav.reshape(-1)
    offs = offsets.reshape(-1).astype(jnp.int32)
    out = _impl(wav_flat, offs)
    return out.reshape(_B, _S, _C, _L)


# speedup vs baseline: 1.8549x; 1.0444x over previous
import functools

import jax
import jax.numpy as jnp
from jax import lax
from jax.experimental import pallas as pl
from jax.experimental.pallas import tpu as pltpu
from jax.experimental.pallas import tpu_sc as plsc

_SHIFT = 44100
_B, _S, _C, _T = 8, 4, 2, 441000
_L = _T - _SHIFT
_NW = 32
_IN_ROW = _C * _T
_OUT_ROW = _C * _L
_CH = 32192
_NFULL = _OUT_ROW // _CH
_TAIL = _OUT_ROW - _NFULL * _CH
_NCH = _NFULL + 1
_JSTR = _L // _CH
_PA = _L - _JSTR * _CH
_PB = _CH - _PA
_BUF = _CH + 48


def _round16(x):
    return (x + 15) // 16 * 16


def _shift_loop(dst, src, rem, length, dst_base):
    nit = (length + 63) // 64

    def body(i, _):
        base = i * 64
        for k in range(4):
            dst[pl.ds(dst_base + base + k * 16, 16)] = (
                src[pl.ds(rem + base + k * 16, 16)])
        return 0

    lax.fori_loop(0, nit, body, 0)


def _impl(wav_flat, offs):
    mesh = plsc.VectorSubcoreMesh(core_axis_name="c", subcore_axis_name="s")

    @functools.partial(
        pl.kernel,
        mesh=mesh,
        out_type=jax.ShapeDtypeStruct((_NW * _OUT_ROW,), jnp.float32),
        scratch_types=[
            pltpu.VMEM((48,), jnp.int32),
            pltpu.VMEM((_BUF,), jnp.float32),
            pltpu.VMEM((_BUF,), jnp.float32),
            pltpu.VMEM((_BUF,), jnp.float32),
            pltpu.VMEM((_BUF,), jnp.float32),
            pltpu.SemaphoreType.DMA,
            pltpu.SemaphoreType.DMA,
            pltpu.SemaphoreType.DMA,
            pltpu.SemaphoreType.DMA,
            pltpu.SemaphoreType.DMA,
            pltpu.SemaphoreType.DMA,
            pltpu.SemaphoreType.DMA,
        ],
    )
    def k(wav_hbm, off_hbm, out_hbm, off_v, b0, b1, b2, strb,
          g0, g1, g2, s0, s1, s2, gstr):
        cid = lax.axis_index("c")
        sid = lax.axis_index("s")
        w = cid * 16 + sid

        pltpu.sync_copy(off_hbm.at[pl.ds(0, 32)], off_v.at[pl.ds(0, 32)])
        off = off_v[pl.ds(w, 16)][0]

        r0 = off % 8
        a0 = off - r0
        r1 = (off + _SHIFT) % 8
        a1 = off + _SHIFT - r1

        wbase_in = w * _IN_ROW
        wbase_out = w * _OUT_ROW
        slots = (b0, b1, b2)
        gsem = (g0, g1, g2)
        ssem = (s0, s1, s2)

        def chunk_len(j):
            return _CH if j < _NFULL else _TAIL

        def issue_gather(j):
            slot = slots[j % 3]
            glen = _round16(chunk_len(j) + 8)
            if j != _JSTR:
                if j < _JSTR:
                    start = wbase_in + a0 + j * _CH
                else:
                    start = wbase_in + a1 + j * _CH
                return (pltpu.async_copy(
                    wav_hbm.at[pl.ds(pl.multiple_of(start, 8), glen)],
                    slot.at[pl.ds(0, glen)], gsem[j % 3]),)
            ga = _round16(_PA + 8)
            gb = _round16(_PB + 8)
            ha = pltpu.async_copy(
                wav_hbm.at[pl.ds(pl.multiple_of(wbase_in + a0 + j * _CH, 8),
                                 ga)],
                slot.at[pl.ds(0, ga)], gsem[j % 3])
            hb = pltpu.async_copy(
                wav_hbm.at[pl.ds(pl.multiple_of(wbase_in + a0 + _T, 8), gb)],
                strb.at[pl.ds(0, gb)], gstr)
            return (ha, hb)

        def realign(j):
            slot = slots[j % 3]
            if j != _JSTR:
                rem = r0 if j < _JSTR else r1

                @pl.when(rem != 0)
                def _():
                    _shift_loop(slot, slot, rem, chunk_len(j), 0)
            else:
                @pl.when(r0 != 0)
                def _():
                    _shift_loop(slot, slot, r0, _PA, 0)
                _shift_loop(slot, strb, r0, _PB, _PA)

        def issue_scatter(j):
            slot = slots[j % 3]
            length = chunk_len(j)
            return pltpu.async_copy(
                slot.at[pl.ds(0, length)],
                out_hbm.at[pl.ds(wbase_out + j * _CH, length)], ssem[j % 3])

        gh = [None] * _NCH
        sh = [None] * _NCH
        gh[0] = issue_gather(0)
        gh[1] = issue_gather(1)
        for j in range(_NCH):
            for h in gh[j]:
                h.wait()
            realign(j)
            sh[j] = issue_scatter(j)
            if j + 2 < _NCH:
                if j - 1 >= 0:
                    sh[j - 1].wait()
                    sh[j - 1] = None
                gh[j + 2] = issue_gather(j + 2)
        for h in sh:
            if h is not None:
                h.wait()

    return k(wav_flat, offs)


def kernel(wav, offsets):
    wav_flat = wav.reshape(-1)
    offs = offsets.reshape(-1).astype(jnp.int32)
    out = _impl(wav_flat, offs)
    return out.reshape(_B, _S, _C, _L)


# speedup vs baseline: 11.8535x; 6.3904x over previous
import functools

import jax
import jax.numpy as jnp
from jax import lax
from jax.experimental import pallas as pl
from jax.experimental.pallas import tpu as pltpu
from jax.experimental.pallas import tpu_sc as plsc

_SHIFT = 44100
_B, _S, _C, _T = 8, 4, 2, 441000
_L = _T - _SHIFT
_NW = 32
_LT = (_L // 128) * 128
_LREM = _L - _LT
_M = 21376
_NFULL = _LT // _M
_TAILC = _LT - _NFULL * _M
_NCH = _NFULL + 1


def _rotate(a, b, s, idxvec, selmask):
    src = jnp.where(selmask, b, a)
    return jnp.take_along_axis(src, idxvec, axis=0, mode="promise_in_bounds")


def _sc_impl(wav3, offs):
    mesh = plsc.VectorSubcoreMesh(core_axis_name="c", subcore_axis_name="s")

    @functools.partial(
        pl.kernel,
        mesh=mesh,
        out_type=[
            jax.ShapeDtypeStruct((_NW, _C, _L), jnp.float32),
            jax.ShapeDtypeStruct((_NW, _C, _LT + 128), jnp.float32),
        ],
        scratch_types=[
            pltpu.VMEM((48,), jnp.int32),
            pltpu.VMEM((_C, _M + 128), jnp.float32),
            pltpu.VMEM((_C, _M + 128), jnp.float32),
            pltpu.VMEM((_C, _M + 128), jnp.float32),
            pltpu.VMEM((_C, 256), jnp.float32),
            pltpu.SemaphoreType.DMA,
            pltpu.SemaphoreType.DMA,
            pltpu.SemaphoreType.DMA,
            pltpu.SemaphoreType.DMA,
            pltpu.SemaphoreType.DMA,
            pltpu.SemaphoreType.DMA,
            pltpu.SemaphoreType.DMA,
            pltpu.SemaphoreType.DMA,
        ],
    )
    def k(wav_hbm, off_hbm, out_hbm, tails_hbm, off_v, b0, b1, b2, traw,
          g0, g1, g2, s0, s1, s2, tg, ts):
        cid = lax.axis_index("c")
        sid = lax.axis_index("s")
        w = cid * 16 + sid

        pltpu.sync_copy(off_hbm.at[pl.ds(0, 32)], off_v.at[pl.ds(0, 32)])
        off = off_v[pl.ds(w, 16)][0]

        col0 = (off // 128) * 128
        phi = off - col0
        s = phi % 16
        phi16 = pl.multiple_of(phi - s, 16)
        lanes = lax.iota(jnp.int32, 16)
        idxvec = (lanes + s) & 15
        selmask = lanes < s

        slots = (b0, b1, b2)
        gsem = (g0, g1, g2)
        ssem = (s0, s1, s2)

        def chunk_cols(j):
            return _M if j < _NFULL else _TAILC

        def issue_gather(j):
            mlen = chunk_cols(j)
            return pltpu.async_copy(
                wav_hbm.at[w, :, pl.ds(pl.multiple_of(col0 + j * _M, 128),
                                       mlen + 128)],
                slots[j % 3].at[:, pl.ds(0, mlen + 128)], gsem[j % 3])

        def realign(buf, mlen):
            nvec = mlen // 16
            for c in range(_C):
                def body(i, a):
                    b = buf[c, pl.ds(phi16 + i * 16 + 16, 16)]
                    buf[c, pl.ds(i * 16, 16)] = _rotate(a, b, s, idxvec,
                                                        selmask)
                    return b
                a0 = buf[c, pl.ds(phi16, 16)]
                lax.fori_loop(0, nvec, body, a0)

        def issue_scatter(j):
            mlen = chunk_cols(j)
            return pltpu.async_copy(
                slots[j % 3].at[:, pl.ds(0, mlen)],
                out_hbm.at[w, :, pl.ds(j * _M, mlen)], ssem[j % 3])

        tail_h = pltpu.async_copy(
            wav_hbm.at[w, :, pl.ds(pl.multiple_of(col0 + _LT, 128), 256)],
            traw, tg)

        gh = [None] * _NCH
        sh = [None] * _NCH
        gh[0] = issue_gather(0)
        gh[1] = issue_gather(1)
        for j in range(_NCH):
            gh[j].wait()
            realign(slots[j % 3], chunk_cols(j))
            sh[j] = issue_scatter(j)
            if j + 2 < _NCH:
                if j - 1 >= 0:
                    sh[j - 1].wait()
                    sh[j - 1] = None
                gh[j + 2] = issue_gather(j + 2)

        tail_h.wait()
        realign(traw, 128)
        pltpu.async_copy(traw.at[:, pl.ds(0, 128)],
                         tails_hbm.at[w, :, pl.ds(_LT, 128)], ts).wait()
        for h in sh:
            if h is not None:
                h.wait()

    return k(wav3, offs)


def _tc_patch(main, tails):
    def patch(main_any, tails_ref, out_ref):
        del main_any
        out_ref[...] = tails_ref[...]

    return pl.pallas_call(
        patch,
        grid=(1,),
        in_specs=[
            pl.BlockSpec(memory_space=pl.ANY),
            pl.BlockSpec((_NW, _C, 128), lambda i: (0, 0, _LT // 128)),
        ],
        out_specs=pl.BlockSpec((_NW, _C, 128), lambda i: (0, 0, _LT // 128)),
        out_shape=jax.ShapeDtypeStruct((_NW, _C, _L), jnp.float32),
        input_output_aliases={0: 0},
    )(main, tails)


def kernel(wav, offsets):
    wav3 = wav.reshape(_NW, _C, _T)
    offs = offsets.reshape(_NW).astype(jnp.int32)
    main, tails = _sc_impl(wav3, offs)
    out = _tc_patch(main, tails)
    return out.reshape(_B, _S, _C, _L)
